# baseline (speedup 1.0000x reference)
import functools

import jax
import jax.numpy as jnp
from jax import lax
from jax.experimental import pallas as pl
from jax.experimental.pallas import tpu as pltpu
from jax.experimental.pallas import tpu_sc as plsc

_NUM_CODES = 512
_DIM = 32
_COMMIT = 0.25
_BLOCK_ROWS = 2048


def _vq_body(x_ref, c2_ref, cn_ref, idx_ref, loss_ref):
    i = pl.program_id(0)
    x = x_ref[...]
    c2 = c2_ref[...]
    cn = cn_ref[...]
    sq = x * x
    t = sq[:, :16] + sq[:, 16:]
    t = t[:, :8] + t[:, 8:]
    t = t[:, :4] + t[:, 4:]
    t = t[:, :2] + t[:, 2:]
    xn = t[:, :1] + t[:, 1:]
    scores2 = jax.lax.dot_general(
        x, c2, (((1,), (1,)), ((), ())),
        preferred_element_type=jnp.float32)
    dist = xn + cn - scores2
    minv = jnp.min(dist, axis=1)
    iota = jax.lax.broadcasted_iota(jnp.int32, dist.shape, 1)
    is_min = dist == minv[:, None]
    idx_ref[...] = jnp.min(
        jnp.where(is_min, iota, dist.shape[1]), axis=1)

    @pl.when(i == 0)
    def _init():
        loss_ref[0, 0] = 0.0

    loss_ref[0, 0] += jnp.sum(minv)


def _tc_argmin(flat, c2, cn):
    n, d = flat.shape
    grid = n // _BLOCK_ROWS
    return pl.pallas_call(
        _vq_body,
        grid=(grid,),
        in_specs=[
            pl.BlockSpec((_BLOCK_ROWS, d), lambda i: (i, 0)),
            pl.BlockSpec((_NUM_CODES, d), lambda i: (0, 0)),
            pl.BlockSpec((1, _NUM_CODES), lambda i: (0, 0)),
        ],
        out_specs=[
            pl.BlockSpec((_BLOCK_ROWS,), lambda i: (i,)),
            pl.BlockSpec(memory_space=pltpu.SMEM),
        ],
        out_shape=[
            jax.ShapeDtypeStruct((n,), jnp.int32),
            jax.ShapeDtypeStruct((1, 1), jnp.float32),
        ],
    )(flat, c2, cn)


_GROW = 256


def _make_sc_gather(n, dp):
    info = plsc.get_sparse_core_info()
    nw = info.num_cores * info.num_subcores
    chunks_per_w = n // (nw * _GROW)
    mesh = plsc.VectorSubcoreMesh(core_axis_name="c", subcore_axis_name="s")

    rows_per_w = n // nw

    @functools.partial(
        pl.kernel, mesh=mesh,
        out_type=jax.ShapeDtypeStruct((n, dp), jnp.float32),
        scratch_types=[
            pltpu.VMEM((rows_per_w,), jnp.int32),
            pltpu.VMEM((_GROW, dp), jnp.float32),
            pltpu.VMEM((_GROW, dp), jnp.float32),
            pltpu.VMEM_SHARED((_NUM_CODES, 128), jnp.float32),
            pltpu.SemaphoreType.DMA,
            pltpu.SemaphoreType.DMA,
        ],
    )
    def gather(table_hbm, idx_hbm, out_hbm, idx_v, rows_a, rows_b,
               table_sh, sg, sw):
        sid = lax.axis_index("s")
        wid = sid * info.num_cores + lax.axis_index("c")
        wbase = wid * rows_per_w

        @pl.when(sid == 0)
        def _stage():
            pltpu.sync_copy(table_hbm, table_sh)

        pltpu.sync_copy(idx_hbm.at[pl.ds(wbase, rows_per_w)], idx_v)
        plsc.subcore_barrier()
        bufs = (rows_a, rows_b)
        g = pltpu.async_copy(
            table_sh.at[idx_v.at[pl.ds(0, _GROW)]], bufs[0], sg)
        writes = []
        for k in range(chunks_per_w):
            g.wait()
            writes.append(pltpu.async_copy(
                bufs[k % 2], out_hbm.at[pl.ds(wbase + k * _GROW, _GROW)], sw))
            if k + 1 < chunks_per_w:
                if k >= 1:
                    writes[k - 1].wait()
                g = pltpu.async_copy(
                    table_sh.at[idx_v.at[pl.ds((k + 1) * _GROW, _GROW)]],
                    bufs[(k + 1) % 2], sg)
        writes[-1].wait()

    return gather


def kernel(z, codebook):
    b0, b1, d = z.shape
    n = b0 * b1
    flat = z.reshape(n, d)
    cn = jnp.sum(codebook ** 2, axis=1)[None, :]
    c2 = codebook * 2.0
    idx, loss_sum = _tc_argmin(flat, c2, cn)
    table_p = jnp.pad(codebook, ((0, 0), (0, 128 - d)))
    qp = _make_sc_gather(n, 128)(table_p, idx)
    q = qp[:, :d].reshape(z.shape)
    vq_loss = loss_sum[0, 0] * ((1.0 + _COMMIT) / (n * d))
    return q, vq_loss


# speedup vs baseline: 1.1788x; 1.1788x over previous
import functools

import jax
import jax.numpy as jnp
from jax import lax
from jax.experimental import pallas as pl
from jax.experimental.pallas import tpu as pltpu
from jax.experimental.pallas import tpu_sc as plsc

_NUM_CODES = 512
_DIM = 32
_COMMIT = 0.25
_BLOCK_ROWS = 4096


def _vq_body(x_ref, c2_ref, xn_ref, cn_ref, idx_ref, loss_ref):
    i = pl.program_id(0)
    x = x_ref[...]
    c2 = c2_ref[...]
    xn = xn_ref[...]
    cn = cn_ref[...]
    scores2 = jax.lax.dot_general(
        x, c2, (((1,), (1,)), ((), ())),
        preferred_element_type=jnp.float32)
    dist = xn + cn - scores2
    minv = jnp.min(dist, axis=1)
    iota = jax.lax.broadcasted_iota(jnp.int32, dist.shape, 1)
    is_min = dist == minv[:, None]
    idx_ref[...] = jnp.min(
        jnp.where(is_min, iota, dist.shape[1]), axis=1)

    @pl.when(i == 0)
    def _init():
        loss_ref[0, 0] = 0.0

    loss_ref[0, 0] += jnp.sum(minv)


def _tc_argmin(flat, c2, xn, cn):
    n, d = flat.shape
    grid = n // _BLOCK_ROWS
    return pl.pallas_call(
        _vq_body,
        grid=(grid,),
        in_specs=[
            pl.BlockSpec((_BLOCK_ROWS, d), lambda i: (i, 0)),
            pl.BlockSpec((_NUM_CODES, d), lambda i: (0, 0)),
            pl.BlockSpec((_BLOCK_ROWS, 1), lambda i: (i, 0)),
            pl.BlockSpec((1, _NUM_CODES), lambda i: (0, 0)),
        ],
        out_specs=[
            pl.BlockSpec((_BLOCK_ROWS,), lambda i: (i,)),
            pl.BlockSpec(memory_space=pltpu.SMEM),
        ],
        out_shape=[
            jax.ShapeDtypeStruct((n,), jnp.int32),
            jax.ShapeDtypeStruct((1, 1), jnp.float32),
        ],
    )(flat, c2, xn, cn)


_GROW = 256


def _make_sc_gather(n, dp):
    info = plsc.get_sparse_core_info()
    nw = info.num_cores * info.num_subcores
    chunks_per_w = n // (nw * _GROW)
    mesh = plsc.VectorSubcoreMesh(core_axis_name="c", subcore_axis_name="s")

    rows_per_w = n // nw

    @functools.partial(
        pl.kernel, mesh=mesh,
        out_type=jax.ShapeDtypeStruct((n, dp), jnp.float32),
        scratch_types=[
            pltpu.VMEM((rows_per_w,), jnp.int32),
            pltpu.VMEM((_GROW, dp), jnp.float32),
            pltpu.VMEM((_GROW, dp), jnp.float32),
            pltpu.VMEM_SHARED((_NUM_CODES, 128), jnp.float32),
            pltpu.SemaphoreType.DMA,
            pltpu.SemaphoreType.DMA,
        ],
    )
    def gather(table_hbm, idx_hbm, out_hbm, idx_v, rows_a, rows_b,
               table_sh, sg, sw):
        sid = lax.axis_index("s")
        wid = sid * info.num_cores + lax.axis_index("c")
        wbase = wid * rows_per_w

        @pl.when(sid == 0)
        def _stage():
            pltpu.sync_copy(table_hbm, table_sh)

        pltpu.sync_copy(idx_hbm.at[pl.ds(wbase, rows_per_w)], idx_v)
        plsc.subcore_barrier()
        bufs = (rows_a, rows_b)
        g = pltpu.async_copy(
            table_sh.at[idx_v.at[pl.ds(0, _GROW)]], bufs[0], sg)
        writes = []
        for k in range(chunks_per_w):
            g.wait()
            writes.append(pltpu.async_copy(
                bufs[k % 2], out_hbm.at[pl.ds(wbase + k * _GROW, _GROW)], sw))
            if k + 1 < chunks_per_w:
                if k >= 1:
                    writes[k - 1].wait()
                g = pltpu.async_copy(
                    table_sh.at[idx_v.at[pl.ds((k + 1) * _GROW, _GROW)]],
                    bufs[(k + 1) % 2], sg)
        writes[-1].wait()

    return gather


def kernel(z, codebook):
    b0, b1, d = z.shape
    n = b0 * b1
    flat = z.reshape(n, d)
    xn = jnp.sum(flat ** 2, axis=1, keepdims=True)
    cn = jnp.sum(codebook ** 2, axis=1)[None, :]
    c2 = codebook * 2.0
    idx, loss_sum = _tc_argmin(flat, c2, xn, cn)
    table_p = jnp.pad(codebook, ((0, 0), (0, 128 - d)))
    qp = _make_sc_gather(n, 128)(table_p, idx)
    q = qp[:, :d].reshape(z.shape)
    vq_loss = loss_sum[0, 0] * ((1.0 + _COMMIT) / (n * d))
    return q, vq_loss


# speedup vs baseline: 1.2777x; 1.0839x over previous
import functools

import jax
import jax.numpy as jnp
from jax import lax
from jax.experimental import pallas as pl
from jax.experimental.pallas import tpu as pltpu
from jax.experimental.pallas import tpu_sc as plsc

_NUM_CODES = 512
_DIM = 32
_COMMIT = 0.25
_BLOCK_ROWS = 4096


def _vq_body(x_ref, c2_ref, xn_ref, cn_ref, idx_ref, loss_ref):
    i = pl.program_id(0)
    x = x_ref[...]
    c2 = c2_ref[...]
    xn = xn_ref[...]
    cn = cn_ref[...]
    scores2 = jax.lax.dot_general(
        x, c2, (((1,), (1,)), ((), ())),
        preferred_element_type=jnp.float32)
    lanes = 128
    ntiles = _NUM_CODES // lanes
    runmin = (xn + cn[:, :lanes]) - scores2[:, :lanes]
    runtile = jnp.zeros(runmin.shape, jnp.int32)
    for t in range(1, ntiles):
        d = (xn + cn[:, t * lanes:(t + 1) * lanes]) \
            - scores2[:, t * lanes:(t + 1) * lanes]
        pred = d < runmin
        runtile = jnp.where(pred, t, runtile)
        runmin = jnp.where(pred, d, runmin)
    minv = jnp.min(runmin, axis=1)
    lane_iota = jax.lax.broadcasted_iota(jnp.int32, runmin.shape, 1)
    runidx = runtile * lanes + lane_iota
    is_min = runmin == minv[:, None]
    idx_ref[...] = jnp.min(
        jnp.where(is_min, runidx, _NUM_CODES), axis=1)

    @pl.when(i == 0)
    def _init():
        loss_ref[0, 0] = 0.0

    loss_ref[0, 0] += jnp.sum(minv)


def _tc_argmin(flat, c2, xn, cn):
    n, d = flat.shape
    grid = n // _BLOCK_ROWS
    return pl.pallas_call(
        _vq_body,
        grid=(grid,),
        in_specs=[
            pl.BlockSpec((_BLOCK_ROWS, d), lambda i: (i, 0)),
            pl.BlockSpec((_NUM_CODES, d), lambda i: (0, 0)),
            pl.BlockSpec((_BLOCK_ROWS, 1), lambda i: (i, 0)),
            pl.BlockSpec((1, _NUM_CODES), lambda i: (0, 0)),
        ],
        out_specs=[
            pl.BlockSpec((_BLOCK_ROWS,), lambda i: (i,)),
            pl.BlockSpec(memory_space=pltpu.SMEM),
        ],
        out_shape=[
            jax.ShapeDtypeStruct((n,), jnp.int32),
            jax.ShapeDtypeStruct((1, 1), jnp.float32),
        ],
    )(flat, c2, xn, cn)


_GROW = 256


def _make_sc_gather(n, dp):
    info = plsc.get_sparse_core_info()
    nw = info.num_cores * info.num_subcores
    chunks_per_w = n // (nw * _GROW)
    mesh = plsc.VectorSubcoreMesh(core_axis_name="c", subcore_axis_name="s")

    rows_per_w = n // nw

    @functools.partial(
        pl.kernel, mesh=mesh,
        out_type=jax.ShapeDtypeStruct((n, dp), jnp.float32),
        scratch_types=[
            pltpu.VMEM((rows_per_w,), jnp.int32),
            pltpu.VMEM((_GROW, dp), jnp.float32),
            pltpu.VMEM((_GROW, dp), jnp.float32),
            pltpu.VMEM_SHARED((_NUM_CODES, 128), jnp.float32),
            pltpu.SemaphoreType.DMA,
            pltpu.SemaphoreType.DMA,
        ],
    )
    def gather(table_hbm, idx_hbm, out_hbm, idx_v, rows_a, rows_b,
               table_sh, sg, sw):
        sid = lax.axis_index("s")
        wid = sid * info.num_cores + lax.axis_index("c")
        wbase = wid * rows_per_w

        @pl.when(sid == 0)
        def _stage():
            pltpu.sync_copy(table_hbm, table_sh)

        pltpu.sync_copy(idx_hbm.at[pl.ds(wbase, rows_per_w)], idx_v)
        plsc.subcore_barrier()
        bufs = (rows_a, rows_b)
        g = pltpu.async_copy(
            table_sh.at[idx_v.at[pl.ds(0, _GROW)]], bufs[0], sg)
        writes = []
        for k in range(chunks_per_w):
            g.wait()
            writes.append(pltpu.async_copy(
                bufs[k % 2], out_hbm.at[pl.ds(wbase + k * _GROW, _GROW)], sw))
            if k + 1 < chunks_per_w:
                if k >= 1:
                    writes[k - 1].wait()
                g = pltpu.async_copy(
                    table_sh.at[idx_v.at[pl.ds((k + 1) * _GROW, _GROW)]],
                    bufs[(k + 1) % 2], sg)
        writes[-1].wait()

    return gather


def kernel(z, codebook):
    b0, b1, d = z.shape
    n = b0 * b1
    flat = z.reshape(n, d)
    xn = jnp.sum(flat ** 2, axis=1, keepdims=True)
    cn = jnp.sum(codebook ** 2, axis=1)[None, :]
    c2 = codebook * 2.0
    idx, loss_sum = _tc_argmin(flat, c2, xn, cn)
    table_p = jnp.pad(codebook, ((0, 0), (0, 128 - d)))
    qp = _make_sc_gather(n, 128)(table_p, idx)
    q = qp[:, :d].reshape(z.shape)
    vq_loss = loss_sum[0, 0] * ((1.0 + _COMMIT) / (n * d))
    return q, vq_loss
